# Initial kernel scaffold; baseline (speedup 1.0000x reference)
import jax
import jax.numpy as jnp
from jax.experimental import pallas as pl


def kernel(x, edge_index, batch, W0, b0, W1, b1, W2, b2, W3, b3, encW0, encb0, encW1, encb1, decW0, decb0, decW1, decb1):
    raise NotImplementedError("write your pallas kernel here")


# speedup vs baseline: 4.0421x; 4.0421x over previous
import functools
import math

import jax
import jax.numpy as jnp
from jax import lax
from jax.experimental import pallas as pl
from jax.experimental.pallas import tpu as pltpu
from jax.experimental.pallas import tpu_sc as plsc

NC = 2
NS = 16
NW = NC * NS
CH = 128
NB = 1280


def _prop_body(y_hbm, srcs_hbm, dsts_hbm, zeros_hbm, z2_hbm, idx0, idx1,
               dst_v, rows0, rows1, si0, si1, sg0, sg1, z_sh):
  c = lax.axis_index("c")
  s = lax.axis_index("s")
  wid = c * NS + s
  npad = z_sh.shape[0]
  npt = npad // NS
  nzch = npt // CH
  nch = dst_v.shape[0]
  pltpu.sync_copy(dsts_hbm.at[wid], dst_v)
  pltpu.sync_copy(zeros_hbm, rows0)
  for j in range(nzch):
    pltpu.sync_copy(rows0, z_sh.at[pl.ds((s * nzch + j) * CH, CH)])
  plsc.subcore_barrier()

  pltpu.async_copy(srcs_hbm.at[wid, 0], idx0, si0)
  pltpu.async_copy(srcs_hbm.at[wid, 1], idx1, si1)
  pltpu.make_async_copy(srcs_hbm.at[wid, 0], idx0, si0).wait()
  pltpu.async_copy(y_hbm.at[idx0], rows0, sg0)

  def body(jh, _):
    j = jh * 2
    pltpu.make_async_copy(srcs_hbm.at[wid, 0], idx1, si1).wait()
    pltpu.async_copy(y_hbm.at[idx1], rows1, sg1)
    pltpu.make_async_copy(y_hbm.at[idx0], rows0, sg0).wait()
    pltpu.async_copy(srcs_hbm.at[wid, j + 2], idx0, si0)
    pltpu.sync_copy(rows0, z_sh.at[dst_v.at[j]], add=True)
    pltpu.make_async_copy(srcs_hbm.at[wid, 0], idx0, si0).wait()
    pltpu.async_copy(y_hbm.at[idx0], rows0, sg0)
    pltpu.make_async_copy(y_hbm.at[idx1], rows1, sg1).wait()
    pltpu.async_copy(srcs_hbm.at[wid, j + 3], idx1, si1)
    pltpu.sync_copy(rows1, z_sh.at[dst_v.at[j + 1]], add=True)
    return 0

  lax.fori_loop(0, nch // 2, body, 0)
  pltpu.make_async_copy(y_hbm.at[idx0], rows0, sg0).wait()
  pltpu.make_async_copy(srcs_hbm.at[wid, 0], idx1, si1).wait()
  plsc.subcore_barrier()
  pltpu.sync_copy(z_sh.at[pl.ds(s * npt, npt)],
                  z2_hbm.at[c, pl.ds(s * npt, npt)])


@functools.lru_cache(maxsize=None)
def _make_prop(npad, nch, d):
  npt = npad // NS
  mesh = plsc.VectorSubcoreMesh(core_axis_name="c", subcore_axis_name="s")
  return pl.kernel(
      _prop_body,
      out_type=jax.ShapeDtypeStruct((NC, npad, d), jnp.float32),
      mesh=mesh,
      scratch_types=[
          pltpu.VMEM((CH,), jnp.int32),
          pltpu.VMEM((CH,), jnp.int32),
          pltpu.VMEM((nch, CH), jnp.int32),
          pltpu.VMEM((CH, d), jnp.float32),
          pltpu.VMEM((CH, d), jnp.float32),
          pltpu.SemaphoreType.DMA,
          pltpu.SemaphoreType.DMA,
          pltpu.SemaphoreType.DMA,
          pltpu.SemaphoreType.DMA,
          pltpu.VMEM_SHARED((npad, d), jnp.float32),
      ],
  )


def _t0_body(nreal, x_ref, deg2_ref, w_ref, y_ref, dis_ref):
  i = pl.program_id(0)
  deg = deg2_ref[0, :, :1] + deg2_ref[1, :, :1] + 1.0
  rows = lax.broadcasted_iota(jnp.int32, (NB, 1), 0) + i * NB
  dis = jnp.where(rows < nreal, lax.rsqrt(deg), 0.0)
  dis_ref[...] = dis
  xw = jnp.dot(x_ref[...], w_ref[...], preferred_element_type=jnp.float32)
  y_ref[...] = xw * dis


def _tmid_body(has_res, z2_ref, y_ref, dis_ref, b_ref, w_ref, res_ref,
               out_ref, ynext_ref):
  z = z2_ref[0] + z2_ref[1] + y_ref[...]
  dis = dis_ref[...]
  o = jnp.maximum(z * dis + b_ref[...][None, :], 0.0)
  if has_res:
    o = o + res_ref[...]
  out_ref[...] = o
  xw = jnp.dot(o, w_ref[...], preferred_element_type=jnp.float32)
  ynext_ref[...] = xw * dis


def _tfin_body(nblocks, z2_ref, y_ref, dis_ref, b_ref, o0_ref, o1_ref, o2_ref,
               ew0_ref, eb0_ref, ew1_ref, eb1_ref, batch_ref, dw0_ref, db0_ref,
               dw1_ref, db1_ref, out_ref, gsum_ref, cnt_ref):
  i = pl.program_id(0)
  z = z2_ref[0] + z2_ref[1] + y_ref[...]
  o3 = jnp.maximum(z * dis_ref[...] + b_ref[...][None, :], 0.0)
  ew0 = ew0_ref[...]
  d = o0_ref.shape[1]
  l1 = (jnp.dot(o0_ref[...], ew0[0 * d:1 * d], preferred_element_type=jnp.float32)
        + jnp.dot(o1_ref[...], ew0[1 * d:2 * d], preferred_element_type=jnp.float32)
        + jnp.dot(o2_ref[...], ew0[2 * d:3 * d], preferred_element_type=jnp.float32)
        + jnp.dot(o3, ew0[3 * d:4 * d], preferred_element_type=jnp.float32))
  e1 = jnp.maximum(l1 + eb0_ref[...][None, :], 0.0)
  e = jnp.maximum(
      jnp.dot(e1, ew1_ref[...], preferred_element_type=jnp.float32)
      + eb1_ref[...][None, :], 0.0)
  g = gsum_ref.shape[0]
  onehot = (batch_ref[...] == lax.broadcasted_iota(jnp.int32, (1, g), 1)
            ).astype(jnp.float32)
  dn = (((0,), (0,)), ((), ()))
  gs = lax.dot_general(onehot, e, dn, preferred_element_type=jnp.float32)
  cn = lax.dot_general(onehot, jnp.ones_like(e), dn,
                       preferred_element_type=jnp.float32)

  @pl.when(i == 0)
  def _():
    gsum_ref[...] = jnp.zeros_like(gsum_ref)
    cnt_ref[...] = jnp.zeros_like(cnt_ref)

  gsum_ref[...] += gs
  cnt_ref[...] += cn

  @pl.when(i == nblocks - 1)
  def _():
    gm = gsum_ref[...] / jnp.maximum(cnt_ref[...], 1.0)
    dd = jnp.maximum(
        jnp.dot(gm, dw0_ref[...], preferred_element_type=jnp.float32)
        + db0_ref[...][None, :], 0.0)
    out_ref[...] = (jnp.dot(dd, dw1_ref[...], preferred_element_type=jnp.float32)
                    + db1_ref[...][None, :])


def _row_spec(d):
  return pl.BlockSpec((NB, d), lambda i: (i, 0))


def _full_spec(shape):
  n = len(shape)
  return pl.BlockSpec(shape, lambda i, _n=n: (0,) * _n)


def kernel(x, edge_index, batch, W0, b0, W1, b1, W2, b2, W3, b3, encW0, encb0,
           encW1, encb1, decW0, decb0, decW1, decb1):
  n, d = x.shape
  e = edge_index.shape[1]
  g = 16
  unit = math.lcm(NS * CH, NB)
  npad = ((n + 1 + unit - 1) // unit) * unit
  nblocks = npad // NB
  nch = -(-e // (NW * CH))
  nch += nch % 2
  epad = NW * nch * CH

  pad_e = jnp.full((epad - e,), n, dtype=edge_index.dtype)
  src = jnp.concatenate([edge_index[0], pad_e]).reshape(NW, nch, CH)
  dst = jnp.concatenate([edge_index[1], pad_e]).reshape(NW, nch, CH)
  src = jnp.concatenate(
      [src, jnp.full((NW, 2, CH), n, dtype=src.dtype)], axis=1)
  x_pad = jnp.concatenate(
      [x, jnp.zeros((npad - n, d), dtype=x.dtype)], axis=0)
  batch_pad = jnp.concatenate(
      [batch, jnp.full((npad - n,), g, dtype=batch.dtype)]).reshape(npad, 1)
  zrow = jnp.zeros((CH, d), jnp.float32)

  prop = _make_prop(npad, nch, d)

  ones_mat = jnp.ones((npad, d), jnp.float32)
  deg2 = prop(ones_mat, src, dst, zrow)

  t0 = pl.pallas_call(
      functools.partial(_t0_body, n),
      grid=(nblocks,),
      in_specs=[
          _row_spec(d),
          pl.BlockSpec((NC, NB, d), lambda i: (0, i, 0)),
          _full_spec((d, d)),
      ],
      out_specs=[_row_spec(d), _row_spec(1)],
      out_shape=[
          jax.ShapeDtypeStruct((npad, d), jnp.float32),
          jax.ShapeDtypeStruct((npad, 1), jnp.float32),
      ],
  )
  y0, dis = t0(x_pad, deg2, W0)


  def tmid(has_res):
    return pl.pallas_call(
        functools.partial(_tmid_body, has_res),
        grid=(nblocks,),
        in_specs=[
            pl.BlockSpec((NC, NB, d), lambda i: (0, i, 0)),
            _row_spec(d),
            _row_spec(1),
            _full_spec((d,)),
            _full_spec((d, d)),
            _row_spec(d),
        ],
        out_specs=[_row_spec(d), _row_spec(d)],
        out_shape=[
            jax.ShapeDtypeStruct((npad, d), jnp.float32),
            jax.ShapeDtypeStruct((npad, d), jnp.float32),
        ],
    )

  z0 = prop(y0, src, dst, zrow)
  out0, y1 = tmid(False)(z0, y0, dis, b0, W1, y0)
  z1 = prop(y1, src, dst, zrow)
  out1, y2 = tmid(False)(z1, y1, dis, b1, W2, y1)
  z2 = prop(y2, src, dst, zrow)
  out2, y3 = tmid(True)(z2, y2, dis, b2, W3, out0)
  z3 = prop(y3, src, dst, zrow)

  nenc0 = encW0.shape[1]
  nenc1 = encW1.shape[1]
  ndec0 = decW0.shape[1]
  ndec1 = decW1.shape[1]
  tfin = pl.pallas_call(
      functools.partial(_tfin_body, nblocks),
      grid=(nblocks,),
      in_specs=[
          pl.BlockSpec((NC, NB, d), lambda i: (0, i, 0)),
          _row_spec(d),
          _row_spec(1),
          _full_spec((d,)),
          _row_spec(d),
          _row_spec(d),
          _row_spec(d),
          _full_spec((4 * d, nenc0)),
          _full_spec((nenc0,)),
          _full_spec((nenc0, nenc1)),
          _full_spec((nenc1,)),
          _row_spec(1),
          _full_spec((nenc1, ndec0)),
          _full_spec((ndec0,)),
          _full_spec((ndec0, ndec1)),
          _full_spec((ndec1,)),
      ],
      out_specs=pl.BlockSpec((g, ndec1), lambda i: (0, 0)),
      out_shape=jax.ShapeDtypeStruct((g, ndec1), jnp.float32),
      scratch_shapes=[
          pltpu.VMEM((g, nenc1), jnp.float32),
          pltpu.VMEM((g, nenc1), jnp.float32),
      ],
  )
  return tfin(z3, y3, dis, b3, out0, out1, out2, encW0, encb0, encW1, encb1,
              batch_pad, decW0, decb0, decW1, decb1)
